# baseline (speedup 1.0000x reference)
import functools

import jax
import jax.numpy as jnp
from jax import lax
from jax.experimental import pallas as pl
from jax.experimental.pallas import tpu as pltpu
from jax.experimental.pallas import tpu_sc as plsc

_B = 16384
_IN_DIM = 100
_N_BINS = 10
_EMB = 16
_HID = 64
_OUT = 64
_FPAD = 128
_RSTRIDE = 65
_TWORDS = 65536
_BT = 1024

_NW = 32
_BW = _B // _NW
_CS = 128


def _prep_kernel(x_ref, embp_ref, w1s_ref, codes_ref, t2_ref, cmax_scr):
    p = pl.program_id(0)
    i = pl.program_id(1)

    @pl.when(p == 0)
    def _colmax_phase():
        part = jnp.max(jnp.abs(x_ref[...]), axis=0, keepdims=True)

        @pl.when(i == 0)
        def _():
            cmax_scr[...] = part

        @pl.when(i > 0)
        def _():
            cmax_scr[...] = jnp.maximum(cmax_scr[...], part)

    @pl.when((p == 1) & (i == 0))
    def _fold_phase():
        acc = embp_ref[:, :, 0:1] * w1s_ref[0]
        for d in range(1, _EMB):
            acc = acc + embp_ref[:, :, d:d + 1] * w1s_ref[d]
        t2_ref[...] = acc

    @pl.when(p == 1)
    def _codes_phase():
        x = x_ref[...]
        d = cmax_scr[...]
        bins = jnp.clip(x / d * (_N_BINS / 2.0) + _N_BINS / 2.0,
                        0.0, _N_BINS - 1).astype(jnp.int32)
        f_iota = jax.lax.broadcasted_iota(jnp.int32, x.shape, 1)
        codes_ref[...] = (f_iota * _N_BINS + bins) * _RSTRIDE


@functools.partial(
    pl.kernel,
    mesh=plsc.VectorSubcoreMesh(core_axis_name="c", subcore_axis_name="s"),
    compiler_params=pltpu.CompilerParams(needs_layout_passes=False),
    out_type=jax.ShapeDtypeStruct((_B * _HID,), jnp.float32),
    scratch_types=[
        pltpu.VMEM((_CS * _IN_DIM,), jnp.int32),
        pltpu.VMEM((_TWORDS,), jnp.float32),
        pltpu.VMEM((_CS * _HID,), jnp.float32),
    ],
)
def _sc_lookup(codes_hbm, table_hbm, out_hbm, codes_v, table_v, h_v):
    wid = lax.axis_index("s") * 2 + lax.axis_index("c")
    lane = lax.iota(jnp.int32, 16)
    lane_c = lane * _IN_DIM
    lane_h = lane * _HID
    pltpu.sync_copy(table_hbm, table_v)

    def chunk_body(c, carry):
        row0 = wid * _BW + c * _CS
        pltpu.sync_copy(codes_hbm.at[pl.ds(row0 * _IN_DIM, _CS * _IN_DIM)],
                        codes_v)

        def g_body(g, carry2):
            for hhc in range(4):
                def f_body(f, accs):
                    bases = plsc.load_gather(
                        codes_v, [lane_c + (g * (16 * _IN_DIM) + f)])
                    return tuple(
                        accs[p] + plsc.load_gather(
                            table_v, [bases + (hhc * 16 + p)])
                        for p in range(16))

                accs = lax.fori_loop(
                    0, _IN_DIM, f_body,
                    tuple(jnp.zeros((16,), jnp.float32) for _ in range(16)))
                for p in range(16):
                    plsc.store_scatter(
                        h_v, [lane_h + (g * (16 * _HID) + hhc * 16 + p)],
                        accs[p])
            return carry2

        lax.fori_loop(0, _CS // 16, g_body, 0)
        pltpu.sync_copy(h_v, out_hbm.at[pl.ds(row0 * _HID, _CS * _HID)])
        return carry

    lax.fori_loop(0, _BW // _CS, chunk_body, 0)


def _tail_kernel(h_ref, b1_ref, w2t_ref, b2_ref, o_ref):
    h = jnp.maximum(h_ref[...] + b1_ref[...], 0.0)
    out = jax.lax.dot(h, w2t_ref[...], preferred_element_type=jnp.float32)
    o_ref[...] = out + b2_ref[...]


def kernel(X, emb, W1, b1, W2, b2):
    B, IN = X.shape
    G = B // _BT

    embp = jnp.pad(jnp.transpose(emb, (1, 0, 2)),
                   ((0, 0), (0, _FPAD - _IN_DIM), (0, 0)))
    w1s = jnp.pad(W1.T.reshape(_IN_DIM, _EMB, _HID).transpose(1, 0, 2),
                  ((0, 0), (0, _FPAD - _IN_DIM), (0, 0)))

    codes, t2 = pl.pallas_call(
        _prep_kernel,
        grid=(2, G),
        in_specs=[
            pl.BlockSpec((_BT, IN), lambda p, i: (i, 0)),
            pl.BlockSpec((_N_BINS, _FPAD, _EMB), lambda p, i: (0, 0, 0)),
            pl.BlockSpec((_EMB, _FPAD, _HID), lambda p, i: (0, 0, 0)),
        ],
        out_specs=[
            pl.BlockSpec((_BT, IN), lambda p, i: (i * p, 0)),
            pl.BlockSpec((_N_BINS, _FPAD, _HID), lambda p, i: (0, 0, 0)),
        ],
        out_shape=[
            jax.ShapeDtypeStruct((B, IN), jnp.int32),
            jax.ShapeDtypeStruct((_N_BINS, _FPAD, _HID), jnp.float32),
        ],
        scratch_shapes=[pltpu.VMEM((1, IN), jnp.float32)],
    )(X, embp, w1s)

    t3 = jnp.transpose(t2, (1, 0, 2))[:_IN_DIM].reshape(
        _IN_DIM * _N_BINS, _HID)
    t3 = jnp.pad(t3, ((0, 0), (0, _RSTRIDE - _HID))).reshape(-1)
    t3 = jnp.pad(t3, (0, _TWORDS - t3.shape[0]))

    h_pre = _sc_lookup(codes.reshape(-1), t3).reshape(B, _HID)

    BT2 = 4096
    out = pl.pallas_call(
        _tail_kernel,
        grid=(B // BT2,),
        in_specs=[
            pl.BlockSpec((BT2, _HID), lambda i: (i, 0)),
            pl.BlockSpec((1, _HID), lambda i: (0, 0)),
            pl.BlockSpec((_HID, _OUT), lambda i: (0, 0)),
            pl.BlockSpec((1, _OUT), lambda i: (0, 0)),
        ],
        out_specs=pl.BlockSpec((BT2, _OUT), lambda i: (i, 0)),
        out_shape=jax.ShapeDtypeStruct((B, _OUT), jnp.float32),
    )(h_pre, b1.reshape(1, -1), W2.T, b2.reshape(1, -1))
    return out


# speedup vs baseline: 5.4514x; 5.4514x over previous
import functools

import jax
import jax.numpy as jnp
from jax import lax
from jax.experimental import pallas as pl
from jax.experimental.pallas import tpu as pltpu
from jax.experimental.pallas import tpu_sc as plsc

_B = 16384
_IN_DIM = 100
_N_BINS = 10
_EMB = 16
_HID = 64
_OUT = 64
_FPAD = 128
_RSTRIDE = 65
_TWORDS = 65536
_BT = 1024

_B_SC = 4096
_B_TC = _B - _B_SC
_NW = 32
_BW = _B_SC // _NW
_CS = 128


def _prep_kernel(x_ref, embp_ref, w1s_ref, codes_ref, t2_ref, cmax_ref,
                 cmax_scr):
    p = pl.program_id(0)
    i = pl.program_id(1)

    @pl.when(p == 0)
    def _colmax_phase():
        part = jnp.max(jnp.abs(x_ref[...]), axis=0, keepdims=True)

        @pl.when(i == 0)
        def _():
            cmax_scr[...] = part

        @pl.when(i > 0)
        def _():
            cmax_scr[...] = jnp.maximum(cmax_scr[...], part)

    @pl.when((p == 1) & (i == 0))
    def _fold_phase():
        acc = embp_ref[:, :, 0:1] * w1s_ref[0]
        for d in range(1, _EMB):
            acc = acc + embp_ref[:, :, d:d + 1] * w1s_ref[d]
        t2_ref[...] = acc
        cmax_ref[...] = cmax_scr[...]

    @pl.when(p == 1)
    def _codes_phase():
        x = x_ref[...]
        d = cmax_scr[...]
        bins = jnp.clip(x / d * (_N_BINS / 2.0) + _N_BINS / 2.0,
                        0.0, _N_BINS - 1).astype(jnp.int32)
        f_iota = jax.lax.broadcasted_iota(jnp.int32, x.shape, 1)
        codes_ref[...] = (f_iota * _N_BINS + bins) * _RSTRIDE


def _main_kernel(x_ref, cmax_ref, t2_ref, b1_ref, w2t_ref, b2_ref, o_ref,
                 t2bf_scr):
    i = pl.program_id(0)

    @pl.when(i == 0)
    def _():
        t2bf_scr[...] = t2_ref[...].astype(jnp.bfloat16)

    x = x_ref[...]
    d = cmax_ref[...]
    bins = jnp.clip(x / d * (_N_BINS / 2.0) + _N_BINS / 2.0,
                    0.0, _N_BINS - 1).astype(jnp.int32)
    pad = jnp.full((x.shape[0], _FPAD - _IN_DIM), -1, jnp.int32)
    binp = jnp.concatenate([bins, pad], axis=1)
    h = jax.lax.dot((binp == 0).astype(jnp.bfloat16), t2bf_scr[0],
                    preferred_element_type=jnp.float32)
    for n in range(1, _N_BINS):
        h = h + jax.lax.dot((binp == n).astype(jnp.bfloat16), t2bf_scr[n],
                            preferred_element_type=jnp.float32)
    h = jnp.maximum(h + b1_ref[...], 0.0)
    out = jax.lax.dot(h, w2t_ref[...], preferred_element_type=jnp.float32)
    o_ref[...] = out + b2_ref[...]


@functools.partial(
    pl.kernel,
    mesh=plsc.VectorSubcoreMesh(core_axis_name="c", subcore_axis_name="s"),
    compiler_params=pltpu.CompilerParams(needs_layout_passes=False),
    out_type=jax.ShapeDtypeStruct((_B_SC * _HID,), jnp.float32),
    scratch_types=[
        pltpu.VMEM((_CS * _IN_DIM,), jnp.int32),
        pltpu.VMEM((_TWORDS,), jnp.float32),
        pltpu.VMEM((_CS * _HID,), jnp.float32),
    ],
)
def _sc_lookup(codes_hbm, table_hbm, out_hbm, codes_v, table_v, h_v):
    wid = lax.axis_index("s") * 2 + lax.axis_index("c")
    lane = lax.iota(jnp.int32, 16)
    lane_c = lane * _IN_DIM
    lane_h = lane * _HID
    pltpu.sync_copy(table_hbm, table_v)

    def chunk_body(c, carry):
        row0 = _B_TC + wid * _BW + c * _CS
        pltpu.sync_copy(codes_hbm.at[pl.ds(row0 * _IN_DIM, _CS * _IN_DIM)],
                        codes_v)

        def g_body(g, carry2):
            for hhc in range(4):
                def f_body(f, accs):
                    bases = plsc.load_gather(
                        codes_v, [lane_c + (g * (16 * _IN_DIM) + f)])
                    return tuple(
                        accs[p] + plsc.load_gather(
                            table_v, [bases + (hhc * 16 + p)])
                        for p in range(16))

                accs = lax.fori_loop(
                    0, _IN_DIM, f_body,
                    tuple(jnp.zeros((16,), jnp.float32) for _ in range(16)))
                for p in range(16):
                    plsc.store_scatter(
                        h_v, [lane_h + (g * (16 * _HID) + hhc * 16 + p)],
                        accs[p])
            return carry2

        lax.fori_loop(0, _CS // 16, g_body, 0)
        pltpu.sync_copy(
            h_v, out_hbm.at[pl.ds((wid * _BW + c * _CS) * _HID, _CS * _HID)])
        return carry

    lax.fori_loop(0, _BW // _CS, chunk_body, 0)


def _tail_kernel(h_ref, b1_ref, w2t_ref, b2_ref, o_ref):
    h = jnp.maximum(h_ref[...] + b1_ref[...], 0.0)
    out = jax.lax.dot(h, w2t_ref[...], preferred_element_type=jnp.float32)
    o_ref[...] = out + b2_ref[...]


def kernel(X, emb, W1, b1, W2, b2):
    B, IN = X.shape
    G = B // _BT

    embp = jnp.pad(jnp.transpose(emb, (1, 0, 2)),
                   ((0, 0), (0, _FPAD - _IN_DIM), (0, 0)))
    w1s = jnp.pad(W1.T.reshape(_IN_DIM, _EMB, _HID).transpose(1, 0, 2),
                  ((0, 0), (0, _FPAD - _IN_DIM), (0, 0)))

    codes, t2, cmax = pl.pallas_call(
        _prep_kernel,
        grid=(2, G),
        in_specs=[
            pl.BlockSpec((_BT, IN), lambda p, i: (i, 0)),
            pl.BlockSpec((_N_BINS, _FPAD, _EMB), lambda p, i: (0, 0, 0)),
            pl.BlockSpec((_EMB, _FPAD, _HID), lambda p, i: (0, 0, 0)),
        ],
        out_specs=[
            pl.BlockSpec((_BT, IN), lambda p, i: (i * p, 0)),
            pl.BlockSpec((_N_BINS, _FPAD, _HID), lambda p, i: (0, 0, 0)),
            pl.BlockSpec((1, IN), lambda p, i: (0, 0)),
        ],
        out_shape=[
            jax.ShapeDtypeStruct((B, IN), jnp.int32),
            jax.ShapeDtypeStruct((_N_BINS, _FPAD, _HID), jnp.float32),
            jax.ShapeDtypeStruct((1, IN), jnp.float32),
        ],
        scratch_shapes=[pltpu.VMEM((1, IN), jnp.float32)],
    )(X, embp, w1s)

    out_tc = pl.pallas_call(
        _main_kernel,
        grid=(_B_TC // _BT,),
        in_specs=[
            pl.BlockSpec((_BT, IN), lambda i: (i, 0)),
            pl.BlockSpec((1, IN), lambda i: (0, 0)),
            pl.BlockSpec((_N_BINS, _FPAD, _HID), lambda i: (0, 0, 0)),
            pl.BlockSpec((1, _HID), lambda i: (0, 0)),
            pl.BlockSpec((_HID, _OUT), lambda i: (0, 0)),
            pl.BlockSpec((1, _OUT), lambda i: (0, 0)),
        ],
        out_specs=pl.BlockSpec((_BT, _OUT), lambda i: (i, 0)),
        out_shape=jax.ShapeDtypeStruct((_B_TC, _OUT), jnp.float32),
        scratch_shapes=[pltpu.VMEM((_N_BINS, _FPAD, _HID), jnp.bfloat16)],
    )(X, cmax, t2, b1.reshape(1, -1), W2.T, b2.reshape(1, -1))

    t3 = jnp.transpose(t2, (1, 0, 2))[:_IN_DIM].reshape(
        _IN_DIM * _N_BINS, _HID)
    t3 = jnp.pad(t3, ((0, 0), (0, _RSTRIDE - _HID))).reshape(-1)
    t3 = jnp.pad(t3, (0, _TWORDS - t3.shape[0]))
    h_pre = _sc_lookup(codes.reshape(-1), t3).reshape(_B_SC, _HID)

    out_sc = pl.pallas_call(
        _tail_kernel,
        grid=(1,),
        in_specs=[
            pl.BlockSpec((_B_SC, _HID), lambda i: (0, 0)),
            pl.BlockSpec((1, _HID), lambda i: (0, 0)),
            pl.BlockSpec((_HID, _OUT), lambda i: (0, 0)),
            pl.BlockSpec((1, _OUT), lambda i: (0, 0)),
        ],
        out_specs=pl.BlockSpec((_B_SC, _OUT), lambda i: (0, 0)),
        out_shape=jax.ShapeDtypeStruct((_B_SC, _OUT), jnp.float32),
    )(h_pre, b1.reshape(1, -1), W2.T, b2.reshape(1, -1))

    return jnp.concatenate([out_tc, out_sc], axis=0)
